# baseline (speedup 1.0000x reference)
import math

import jax
import jax.numpy as jnp
from jax import lax
from jax.experimental import pallas as pl
from jax.experimental.pallas import tpu as pltpu
from jax.experimental.pallas import tpu_sc as plsc
import numpy as np

_VOCAB = 1000000
_D = 128
_MAX_LEN = 200
_NUM_LAYERS = 6
_B = 4096
_L = 200

_NC = 2
_NS = 16
_NW = _NC * _NS

_CHUNK = _B // _NW
_SCALE = math.sqrt(float(_D))
_LANES = 16
_VPR = _D // _LANES
_UNROLL = 8


def _sincos_table(max_len, d_model):
    pe = np.zeros((max_len, d_model), dtype=np.float32)
    pos = np.arange(max_len, dtype=np.float64)[:, None]
    i = np.arange(0, d_model, 2, dtype=np.float64)
    pe[:, 0::2] = np.sin(pos / np.power(10000.0, 2.0 * i / d_model)).astype(np.float32)
    pe[:, 1::2] = np.cos(pos / np.power(10000.0, 2.0 * (i + 1.0) / d_model)).astype(np.float32)
    return pe


_PE = _sincos_table(_MAX_LEN, _D)
_TE = _sincos_table(_NUM_LAYERS, _D)


_NB = 4


def _body(table_hbm, idx_hbm, pe_hbm, out_hbm,
          idx_all, g0, g1, g2, g3, pe_v, gs0, gs1, gs2, gs3,
          ss0, ss1, ss2, ss3):
    gbuf = [g0, g1, g2, g3]
    gsem, ssem = [gs0, gs1, gs2, gs3], [ss0, ss1, ss2, ss3]

    wid = lax.axis_index("s") * _NC + lax.axis_index("c")
    b0 = wid * _CHUNK
    pltpu.sync_copy(pe_hbm, pe_v)
    pltpu.sync_copy(idx_hbm.at[:, pl.ds(b0, _CHUNK)], idx_all)

    def out_slice(c):
        return out_hbm.at[pl.ds(b0, _CHUNK), pl.ds(c * _D, _D)]

    for g in range(2):
        pltpu.make_async_copy(table_hbm.at[idx_all.at[g]], gbuf[g],
                              gsem[g]).start()

    def chunk(c, g, refill):
        g2 = (g + 2) % _NB
        if refill:
            @pl.when(c + 2 < _L)
            def _():
                @pl.when(c >= 2)
                def _():
                    pltpu.make_async_copy(gbuf[g2], out_slice(c - 2),
                                          ssem[g2]).wait()
                pltpu.make_async_copy(table_hbm.at[idx_all.at[c + 2]],
                                      gbuf[g2], gsem[g2]).start()

        pltpu.make_async_copy(table_hbm.at[idx_all.at[c]], gbuf[g],
                              gsem[g]).wait()

        vp = [pe_v[c, pl.ds(j * _LANES, _LANES)] for j in range(_VPR)]

        def blk(t, carry2):
            for r in range(_UNROLL):
                i = t * _UNROLL + r
                for j in range(_VPR):
                    sl = pl.ds(j * _LANES, _LANES)
                    gbuf[g][i, sl] = gbuf[g][i, sl] * _SCALE + vp[j]
            return carry2

        lax.fori_loop(0, _CHUNK // _UNROLL, blk, 0)

        pltpu.make_async_copy(gbuf[g], out_slice(c), ssem[g]).start()

    def outer(k, carry):
        for u in range(_NB):
            chunk(k * _NB + u, u, refill=True)
        return carry

    lax.fori_loop(0, _L // _NB, outer, 0)

    for c in range(_L - 4, _L):
        pltpu.make_async_copy(gbuf[c % _NB], out_slice(c),
                              ssem[c % _NB]).wait()


def kernel(embedding, layer_idx, embed_weight):
    pe = jnp.asarray(_PE)
    te_row = jnp.take(jnp.asarray(_TE), layer_idx, axis=0)
    pe_c = pe + te_row[None, :]

    idx_t = embedding.astype(jnp.int32).T

    mesh = plsc.VectorSubcoreMesh(core_axis_name="c", subcore_axis_name="s")
    out = pl.kernel(
        _body,
        out_type=jax.ShapeDtypeStruct((_B, _L * _D), jnp.float32),
        mesh=mesh,
        scratch_types=(
            [pltpu.VMEM((_L, _CHUNK), jnp.int32)]
            + [pltpu.VMEM((_CHUNK, _D), jnp.float32)] * _NB
            + [pltpu.VMEM((_MAX_LEN, _D), jnp.float32)]
            + [pltpu.SemaphoreType.DMA] * (2 * _NB)
        ),
    )(embed_weight, idx_t, pe_c)
    return out.reshape(_B, _L, _D)


# speedup vs baseline: 1.9069x; 1.9069x over previous
import math

import jax
import jax.numpy as jnp
from jax import lax
from jax.experimental import pallas as pl
from jax.experimental.pallas import tpu as pltpu
from jax.experimental.pallas import tpu_sc as plsc
import numpy as np

_VOCAB = 1000000
_D = 128
_MAX_LEN = 200
_NUM_LAYERS = 6
_B = 4096
_L = 200

_NC = 2
_NS = 16
_NW = _NC * _NS

_CHUNK = _B // _NW
_SCALE = math.sqrt(float(_D))
_LANES = 16
_VPR = _D // _LANES
_UNROLL = 8


def _sincos_table(max_len, d_model):
    pe = np.zeros((max_len, d_model), dtype=np.float32)
    pos = np.arange(max_len, dtype=np.float64)[:, None]
    i = np.arange(0, d_model, 2, dtype=np.float64)
    pe[:, 0::2] = np.sin(pos / np.power(10000.0, 2.0 * i / d_model)).astype(np.float32)
    pe[:, 1::2] = np.cos(pos / np.power(10000.0, 2.0 * (i + 1.0) / d_model)).astype(np.float32)
    return pe


_PE = _sincos_table(_MAX_LEN, _D)
_TE = _sincos_table(_NUM_LAYERS, _D)


_NB = 2


def _body(table_hbm, idx_hbm, pe_hbm, out_hbm,
          idx_all, g0, g1, o0, o1, pe_v, gs0, gs1, ss0, ss1):
    gbuf, obuf = [g0, g1], [o0, o1]
    gsem, ssem = [gs0, gs1], [ss0, ss1]

    wid = lax.axis_index("s") * _NC + lax.axis_index("c")
    b0 = wid * _CHUNK
    pltpu.sync_copy(pe_hbm, pe_v)
    pltpu.sync_copy(idx_hbm.at[:, pl.ds(b0, _CHUNK)], idx_all)

    for g in range(_NB):
        pltpu.make_async_copy(table_hbm.at[idx_all.at[g]], gbuf[g],
                              gsem[g]).start()

    def chunk(c, g, refill):
        pltpu.make_async_copy(table_hbm.at[idx_all.at[c]], gbuf[g],
                              gsem[g]).wait()

        @pl.when(c >= _NB)
        def _():
            pltpu.make_async_copy(
                obuf[g],
                out_hbm.at[pl.ds(b0, _CHUNK), pl.ds(c - _NB, 1), :],
                ssem[g]).wait()

        vp = [pe_v[c, pl.ds(j * _LANES, _LANES)] for j in range(_VPR)]

        def blk(t, carry2):
            for r in range(_UNROLL):
                i = t * _UNROLL + r
                for j in range(_VPR):
                    sl = pl.ds(j * _LANES, _LANES)
                    obuf[g][i, 0, sl] = gbuf[g][i, sl] * _SCALE + vp[j]
            return carry2

        lax.fori_loop(0, _CHUNK // _UNROLL, blk, 0)

        pltpu.make_async_copy(
            obuf[g], out_hbm.at[pl.ds(b0, _CHUNK), pl.ds(c, 1), :],
            ssem[g]).start()

        if refill:
            @pl.when(c + _NB < _L)
            def _():
                pltpu.make_async_copy(table_hbm.at[idx_all.at[c + _NB]],
                                      gbuf[g], gsem[g]).start()

    def outer(k, carry):
        for u in range(_NB):
            chunk(k * _NB + u, u, refill=True)
        return carry

    lax.fori_loop(0, _L // _NB, outer, 0)

    for c in range(_L - _NB, _L):
        pltpu.make_async_copy(
            obuf[c % _NB], out_hbm.at[pl.ds(b0, _CHUNK), pl.ds(c, 1), :],
            ssem[c % _NB]).wait()


def kernel(embedding, layer_idx, embed_weight):
    pe = jnp.asarray(_PE)
    te_row = jnp.take(jnp.asarray(_TE), layer_idx, axis=0)
    pe_c = pe + te_row[None, :]

    idx_t = embedding.astype(jnp.int32).T

    mesh = plsc.VectorSubcoreMesh(core_axis_name="c", subcore_axis_name="s")
    out = pl.kernel(
        _body,
        out_type=jax.ShapeDtypeStruct((_B, _L, _D), jnp.float32),
        mesh=mesh,
        scratch_types=(
            [pltpu.VMEM((_L, _CHUNK), jnp.int32)]
            + [pltpu.VMEM((_CHUNK, _D), jnp.float32)] * _NB
            + [pltpu.VMEM((_CHUNK, 1, _D), jnp.float32)] * _NB
            + [pltpu.VMEM((_MAX_LEN, _D), jnp.float32)]
            + [pltpu.SemaphoreType.DMA] * (2 * _NB)
        ),
    )(embed_weight, idx_t, pe_c)
    return out
